# Initial kernel scaffold; baseline (speedup 1.0000x reference)
import jax
import jax.numpy as jnp
from jax.experimental import pallas as pl


def kernel(local_x, voxel_x, z, params, local_type, voxel_type, edge_index):
    raise NotImplementedError("write your pallas kernel here")


# speedup vs baseline: 5.3219x; 5.3219x over previous
import functools

import jax
import jax.numpy as jnp
from jax import lax
from jax.experimental import pallas as pl
from jax.experimental.pallas import tpu as pltpu
from jax.experimental.pallas import tpu_sc as plsc

N = 50000
N_TYPES = 10
NPAD = 50048
STRIPE = NPAD // 16
E_RAW = 800000
EP = 802816
EPW = EP // 32
NB = EPW // 128
R = 1000
EPS = 1e-5

_f32 = jnp.float32
_sds = jax.ShapeDtypeStruct


def _sc_mesh():
    return plsc.VectorSubcoreMesh(core_axis_name="c", subcore_axis_name="s")


@functools.lru_cache(maxsize=None)
def _make_deg_kernel():
    @functools.partial(
        pl.kernel,
        mesh=_sc_mesh(),
        out_type=_sds((2, NPAD, 16), _f32),
        compiler_params=pltpu.CompilerParams(use_tc_tiling_on_sc=False),
        scratch_types=[
            pltpu.VMEM((128,), jnp.int32),
            pltpu.VMEM((128, 16), _f32),
            pltpu.VMEM_SHARED((NPAD, 16), _f32),
        ],
    )
    def deg_kernel(dst_hbm, ones_hbm, zeros_hbm, out_hbm, dst_v, ones_v, acc):
        cid = lax.axis_index("c")
        sid = lax.axis_index("s")
        wid = cid * 16 + sid
        pltpu.sync_copy(ones_hbm, ones_v)
        pltpu.sync_copy(zeros_hbm, acc.at[pl.ds(sid * STRIPE, STRIPE)])
        plsc.subcore_barrier()

        def body(j, carry):
            base = wid * EPW + j * 128
            pltpu.sync_copy(dst_hbm.at[pl.ds(base, 128)], dst_v)
            pltpu.sync_copy(ones_v, acc.at[dst_v], add=True)
            return carry

        lax.fori_loop(0, NB, body, 0)
        plsc.subcore_barrier()
        pltpu.sync_copy(acc.at[pl.ds(sid * STRIPE, STRIPE)],
                        out_hbm.at[cid, pl.ds(sid * STRIPE, STRIPE)])

    return deg_kernel


@functools.lru_cache(maxsize=None)
def _make_scatter_kernel(k):
    @functools.partial(
        pl.kernel,
        mesh=_sc_mesh(),
        out_type=_sds((2, NPAD, k, 32), _f32),
        compiler_params=pltpu.CompilerParams(use_tc_tiling_on_sc=False),
        scratch_types=[
            pltpu.VMEM((128,), jnp.int32),
            pltpu.VMEM((128,), jnp.int32),
            pltpu.VMEM((128, 32), _f32),
            pltpu.VMEM_SHARED((NPAD, 32), _f32),
            pltpu.SemaphoreType.DMA,
        ],
    )
    def scat_kernel(hmp_hbm, idx_hbm, dst_hbm, zeros_hbm, out_hbm,
                    idx_v, dst_v, rows_v, acc, sem):
        cid = lax.axis_index("c")
        sid = lax.axis_index("s")
        wid = cid * 16 + sid
        for c in range(k):
            pltpu.sync_copy(zeros_hbm, acc.at[pl.ds(sid * STRIPE, STRIPE)])
            plsc.subcore_barrier()

            def body(j, carry):
                base = wid * EPW + j * 128
                pltpu.sync_copy(idx_hbm.at[pl.ds(c * EP + base, 128)], idx_v)
                pltpu.sync_copy(dst_hbm.at[pl.ds(base, 128)], dst_v)
                pltpu.async_copy(hmp_hbm.at[idx_v], rows_v, sem).wait()
                pltpu.sync_copy(rows_v, acc.at[dst_v], add=True)
                return carry

            lax.fori_loop(0, NB, body, 0)
            plsc.subcore_barrier()
            pltpu.sync_copy(acc.at[pl.ds(sid * STRIPE, STRIPE)],
                            out_hbm.at[cid, pl.ds(sid * STRIPE, STRIPE), c])

    return scat_kernel


def _mm(xs, w, b):
    n = xs[0].shape[0]
    o = w.shape[1]
    nx = len(xs)

    def kern(*refs):
        xrefs = refs[:nx]
        w_ref, b_ref = refs[nx], refs[nx + 1]
        y_ref = refs[nx + 2]
        if nx == 1:
            x = xrefs[0][...]
        else:
            x = jnp.concatenate([r[...] for r in xrefs], axis=1)
        xb = jnp.concatenate([x, jnp.ones((R, 1), _f32)], axis=1)
        wb = jnp.concatenate([w_ref[...], b_ref[...]], axis=0)
        y_ref[...] = jnp.dot(xb, wb, preferred_element_type=_f32,
                             precision=lax.Precision.HIGHEST)

    return pl.pallas_call(
        kern,
        grid=(n // R,),
        in_specs=[pl.BlockSpec((R, x.shape[1]), lambda i: (i, 0)) for x in xs]
        + [pl.BlockSpec(w.shape, lambda i: (0, 0)),
           pl.BlockSpec((1, o), lambda i: (0, 0))],
        out_specs=pl.BlockSpec((R, o), lambda i: (i, 0)),
        out_shape=_sds((n, o), _f32),
    )(*xs, w, b.reshape(1, o))


def _norm_act(y, alpha_v, gamma, beta, leaky, alpha_arr=None):
    n, o = y.shape
    mu = jnp.mean(y, axis=0)
    if alpha_arr is None:
        var = jnp.var(y, axis=0)
    else:
        sh = y - alpha_arr * mu
        var = jnp.mean(sh * sh, axis=0)
    sq = jnp.sqrt(var + EPS)

    def kern(y_ref, mu_ref, sq_ref, a_ref, g_ref, be_ref, o_ref):
        t = (g_ref[...] * (y_ref[...] - a_ref[...] * mu_ref[...])
             / sq_ref[...] + be_ref[...])
        if leaky:
            o_ref[...] = jnp.where(t >= 0, t, 0.2 * t)
        else:
            o_ref[...] = jnp.maximum(t, 0.0)

    return pl.pallas_call(
        kern,
        grid=(n // R,),
        in_specs=[pl.BlockSpec((R, o), lambda i: (i, 0))]
        + [pl.BlockSpec((1, o), lambda i: (0, 0))] * 5,
        out_specs=pl.BlockSpec((R, o), lambda i: (i, 0)),
        out_shape=_sds((n, o), _f32),
    )(y, mu.reshape(1, o), sq.reshape(1, o), alpha_v.reshape(1, o),
      gamma.reshape(1, o), beta.reshape(1, o))


def _dis_from_parts(parts):
    def kern(p_ref, o_ref):
        p = p_ref[...]
        deg = p[0, :, 0:8] + p[1, :, 0:8] + 1.0
        o_ref[...] = 1.0 / jnp.sqrt(deg)

    return pl.pallas_call(
        kern,
        grid=(N // R,),
        in_specs=[pl.BlockSpec((2, R, 16), lambda i: (0, i, 0))],
        out_specs=pl.BlockSpec((R, 8), lambda i: (i, 0)),
        out_shape=_sds((N, 8), _f32),
    )(parts)


def _sel_mm(onehot_v, m, w, b):
    o = w.shape[1]

    def kern(oh_ref, m_ref, w_ref, b_ref, y_ref):
        sel = jnp.dot(oh_ref[...], m_ref[...], preferred_element_type=_f32,
                      precision=lax.Precision.HIGHEST)
        xb = jnp.concatenate([sel, jnp.ones((R, 1), _f32)], axis=1)
        wb = jnp.concatenate([w_ref[...], b_ref[...]], axis=0)
        y_ref[...] = jnp.dot(xb, wb, preferred_element_type=_f32,
                             precision=lax.Precision.HIGHEST)

    return pl.pallas_call(
        kern,
        grid=(N // R,),
        in_specs=[pl.BlockSpec((R, 16), lambda i: (i, 0)),
                  pl.BlockSpec(m.shape, lambda i: (0, 0)),
                  pl.BlockSpec(w.shape, lambda i: (0, 0)),
                  pl.BlockSpec((1, o), lambda i: (0, 0))],
        out_specs=pl.BlockSpec((R, o), lambda i: (i, 0)),
        out_shape=_sds((N, o), _f32),
    )(onehot_v, m, w, b.reshape(1, o))


def _conv_transform(e, w, dis):
    n, _ = e.shape
    o = w.shape[1]

    def kern(e_ref, w_ref, d_ref, o_ref):
        o_ref[...] = jnp.dot(e_ref[...], w_ref[...],
                             preferred_element_type=_f32) * d_ref[...][:, 0:1]

    return pl.pallas_call(
        kern,
        grid=(n // R,),
        in_specs=[pl.BlockSpec((R, e.shape[1]), lambda i: (i, 0)),
                  pl.BlockSpec(w.shape, lambda i: (0, 0)),
                  pl.BlockSpec((R, 8), lambda i: (i, 0))],
        out_specs=pl.BlockSpec((R, o), lambda i: (i, 0)),
        out_shape=_sds((n, o), _f32),
    )(e, w, dis)


def _conv_combine(parts, hmp, dis, b):
    o = hmp.shape[1]

    def kern(p_ref, h_ref, d_ref, b_ref, y_ref):
        p = p_ref[...]
        y_ref[...] = (d_ref[...][:, 0:1] * (p[0] + p[1] + h_ref[...])
                      + b_ref[...])

    return pl.pallas_call(
        kern,
        grid=(N // R,),
        in_specs=[pl.BlockSpec((2, R, o), lambda i: (0, i, 0)),
                  pl.BlockSpec((R, o), lambda i: (i, 0)),
                  pl.BlockSpec((R, 8), lambda i: (i, 0)),
                  pl.BlockSpec((1, o), lambda i: (0, 0))],
        out_specs=pl.BlockSpec((R, o), lambda i: (i, 0)),
        out_shape=_sds((N, o), _f32),
    )(parts, hmp, dis, b.reshape(1, o))


def _head(d, wl, bl, gum):
    o = wl.shape[1]

    def kern(d_ref, w_ref, b_ref, g_ref, lo_ref, lh_ref, ls_ref):
        lo = jnp.dot(jnp.concatenate([d_ref[...], jnp.ones((R, 1), _f32)], 1),
                     jnp.concatenate([w_ref[...], b_ref[...]], 0),
                     preferred_element_type=_f32,
                     precision=lax.Precision.HIGHEST)
        lo_ref[...] = lo
        y = lo + g_ref[...]
        m = jnp.max(y, axis=1, keepdims=True)
        ey = jnp.exp(y - m)
        soft = ey / jnp.sum(ey, axis=1, keepdims=True)
        ls_ref[...] = soft
        mx = jnp.max(soft, axis=1, keepdims=True)
        ii = lax.broadcasted_iota(jnp.int32, (R, o), 1)
        cand = jnp.where(soft == mx, ii, o)
        amin = jnp.min(cand, axis=1, keepdims=True)
        hard = jnp.where(ii == amin, 1.0, 0.0)
        lh_ref[...] = (hard - soft) + soft

    return pl.pallas_call(
        kern,
        grid=(N // R,),
        in_specs=[pl.BlockSpec((R, d.shape[1]), lambda i: (i, 0)),
                  pl.BlockSpec(wl.shape, lambda i: (0, 0)),
                  pl.BlockSpec((1, o), lambda i: (0, 0)),
                  pl.BlockSpec((R, o), lambda i: (i, 0))],
        out_specs=[pl.BlockSpec((R, o), lambda i: (i, 0))] * 3,
        out_shape=[_sds((N, o), _f32)] * 3,
    )(d, wl, bl.reshape(1, o), gum)


def kernel(local_x, voxel_x, z, params, local_type, voxel_type, edge_index):
    src = edge_index[0].astype(jnp.int32)
    dst = edge_index[1].astype(jnp.int32)
    pad = EP - E_RAW
    src_p = jnp.concatenate([src, jnp.zeros((pad,), jnp.int32)])
    dst_p = jnp.concatenate([dst, jnp.full((pad,), NPAD - 1, jnp.int32)])

    zeros16 = jnp.zeros((STRIPE, 16), _f32)
    zeros32 = jnp.zeros((STRIPE, 32), _f32)
    ones16 = jnp.ones((128, 16), _f32)

    deg_parts = _make_deg_kernel()(dst_p, ones16, zeros16)
    dis = _dis_from_parts(deg_parts)

    counts = jnp.bincount(local_type, length=N_TYPES).astype(_f32)
    sums = jax.ops.segment_sum(local_x, local_type, num_segments=N_TYPES)
    means = sums / jnp.clip(counts, 1.0)[:, None]
    m10 = jnp.where((counts > 0)[:, None], means, 0.0)
    m = jnp.zeros((16, 128), _f32).at[:N_TYPES].set(m10)
    onehot_v = (voxel_type[:, None] == jnp.arange(16)[None, :]).astype(_f32)
    (w1, b1, g1, be1), (w2, b2, g2, be2) = params['le']
    ones_o = jnp.ones((w1.shape[1],), _f32)
    y = _sel_mm(onehot_v, m, w1, b1)
    h = _norm_act(y, ones_o, g1, be1, leaky=True)
    y = _mm([h], w2, b2)
    encoded = _norm_act(y, ones_o, g2, be2, leaky=True)

    zs = z[0]
    x = None
    cur = [encoded, voxel_x, zs]
    for (w, b, g, be) in params['mlp']:
        y = _mm(cur, w, b)
        x = _norm_act(y, jnp.ones((w.shape[1],), _f32), g, be, leaky=True)
        cur = [x]

    e = x
    for (w, b, al, ga, be) in params['conv']:
        o = w.shape[1]
        k = o // 32
        hmp = _conv_transform(e, w, dis)
        hmp_rows = hmp.reshape(N * k, 32)
        idx = jnp.concatenate([src_p * k + c for c in range(k)])
        parts = _make_scatter_kernel(k)(hmp_rows, idx, dst_p, zeros32)
        parts = parts.reshape(2, NPAD, o)
        y = _conv_combine(parts, hmp, dis, b)
        e = _norm_act(y, al, ga, be, leaky=False, alpha_arr=al)

    cur = [e, x, encoded, voxel_x, zs]
    d = None
    for (w, b, g, be) in params['dec']:
        y = _mm(cur, w, b)
        d = _norm_act(y, jnp.ones((w.shape[1],), _f32), g, be, leaky=True)
        cur = [d]

    wl, bl = params['dec_final']
    u = jax.random.uniform(jax.random.key(42), (N, wl.shape[1]),
                           minval=1e-6, maxval=1.0 - 1e-6)
    gum = -jnp.log(-jnp.log(u))
    logits, label_hard, label_soft = _head(d, wl, bl, gum)
    return logits, label_hard, label_soft
